# Initial kernel scaffold; baseline (speedup 1.0000x reference)
import jax
import jax.numpy as jnp
from jax.experimental import pallas as pl


def kernel(A, AX, Wr_w, Wr_b, W_w, W_b):
    raise NotImplementedError("write your pallas kernel here")


# speedup vs baseline: 1.0094x; 1.0094x over previous
import functools

import jax
import jax.numpy as jnp
from jax.experimental import pallas as pl
from jax.experimental.pallas import tpu as pltpu

N = 10000
D = 128
BM = 400


def _fused_gcn_kernel(a_ref, ax_ref, wrT_ref, wrb_ref, wT_ref, wb_ref,
                      out_ref, h_ref):
    @pl.when(pl.program_id(0) == 0)
    def _compute_h():
        h = jnp.dot(ax_ref[...], wrT_ref[...],
                    preferred_element_type=jnp.float32) + wrb_ref[...]
        h_ref[...] = jnp.maximum(h, 0.0)

    temp = jnp.dot(a_ref[...], h_ref[...], preferred_element_type=jnp.float32)
    out = jnp.dot(temp, wT_ref[...], preferred_element_type=jnp.float32)
    out_ref[...] = jnp.maximum(out + wb_ref[...], 0.0)


@jax.jit
def _run(A, AX, WrT, Wr_b, WT, W_b):
    grid = (N // BM,)
    out = pl.pallas_call(
        _fused_gcn_kernel,
        grid=grid,
        in_specs=[
            pl.BlockSpec((BM, N), lambda i: (i, 0)),
            pl.BlockSpec((N, D), lambda i: (0, 0)),
            pl.BlockSpec((D, D), lambda i: (0, 0)),
            pl.BlockSpec((1, D), lambda i: (0, 0)),
            pl.BlockSpec((D, D), lambda i: (0, 0)),
            pl.BlockSpec((1, D), lambda i: (0, 0)),
        ],
        out_specs=pl.BlockSpec((BM, D), lambda i: (i, 0)),
        out_shape=jax.ShapeDtypeStruct((N, D), jnp.float32),
        scratch_shapes=[pltpu.VMEM((N, D), jnp.float32)],
        compiler_params=pltpu.CompilerParams(
            dimension_semantics=("arbitrary",),
        ),
    )(A, AX, WrT, Wr_b, WT, W_b)
    return out[None, :, :]


def kernel(A, AX, Wr_w, Wr_b, W_w, W_b):
    return _run(A, AX, Wr_w.T, Wr_b.reshape(1, D), W_w.T, W_b.reshape(1, D))
